# Initial kernel scaffold; baseline (speedup 1.0000x reference)
import jax
import jax.numpy as jnp
from jax.experimental import pallas as pl


def kernel(tokens, tables, Wq, bq, codebook_a, codebook_b, values, Wout, bout, ln_g, ln_b):
    raise NotImplementedError("write your pallas kernel here")


# speedup vs baseline: 2.8387x; 2.8387x over previous
import functools

import jax
import jax.numpy as jnp
from jax import lax
from jax.experimental import pallas as pl
from jax.experimental.pallas import tpu as pltpu
from jax.experimental.pallas import tpu_sc as plsc

_HASH_PRIMES = [2654435761, 2246822519, 3266489917, 2028178513,
                1220703125, 1610612741, 805306457, 402653189]
_P16 = [p & 0xFFFF for p in _HASH_PRIMES]
_WINDOWS = (1, 2, 4, 8, 16, 32, 64, 128)

_DIM = 1024
_NUM_TABLES = 8
_BUCKETS = 65536
_EMBED_DIM = 128
_SUB_KEYS = 512
_TOP_K = 32
_KEY_DIM = 256
_VALUE_DIM = 256

_CNTS = [32 // (i + 1) for i in range(32)]
_NCAND = sum(_CNTS)
_CPAD = 128


def _hash_body(tok_ref, out_ref):
    t = tok_ref[...]
    z = jnp.zeros((t.shape[0], 1), jnp.int32)
    s = t
    acc = jnp.zeros_like(t)
    ti = 0
    for o in range(1, _WINDOWS[-1] + 1):
        s = jnp.concatenate([z, s[:, :-1]], axis=1)
        acc = acc ^ (s * _P16[(o - 1) % 8])
        if o == _WINDOWS[ti]:
            out_ref[ti] = (acc & 0xFFFF) + ti * _BUCKETS
            ti += 1


def _hash_indices(tok32):
    return pl.pallas_call(
        _hash_body,
        out_shape=jax.ShapeDtypeStruct((_NUM_TABLES,) + tok32.shape, jnp.int32),
    )(tok32)


def _sc_gather(table, idx_flat):
    V, D = table.shape
    NI = idx_flat.shape[0]
    NC, NW, CH = 2, 32, 128
    b_per_w = NI // NW
    n_chunks = b_per_w // CH
    mesh = plsc.VectorSubcoreMesh(core_axis_name="c", subcore_axis_name="s")

    @functools.partial(
        pl.kernel, mesh=mesh,
        out_type=jax.ShapeDtypeStruct((NI, D), jnp.float32),
        scratch_types=[
            pltpu.VMEM((CH,), jnp.int32),
            pltpu.VMEM((CH, D), jnp.float32),
            pltpu.SemaphoreType.DMA,
        ],
    )
    def k(table_hbm, idx_hbm, out_hbm, idx_v, rows_v, sem):
        wid = lax.axis_index("s") * jnp.int32(NC) + lax.axis_index("c")
        base = wid * jnp.int32(b_per_w)

        @pl.loop(0, n_chunks)
        def _(ci):
            off = base + ci * jnp.int32(CH)
            pltpu.sync_copy(idx_hbm.at[pl.ds(off, CH)], idx_v)
            pltpu.async_copy(table_hbm.at[idx_v], rows_v, sem).wait()
            pltpu.sync_copy(rows_v, out_hbm.at[pl.ds(off, CH)])

    return k(table, idx_flat)


def _topk32_packed(s):
    n_idx_bits_mask = 511
    lane = lax.broadcasted_iota(jnp.int32, s.shape, 1)
    si = lax.bitcast_convert_type(s, jnp.int32)
    pv = lax.bitcast_convert_type((si & (~n_idx_bits_mask)) | lane, jnp.float32)
    outs = []
    for _ in range(_TOP_K):
        m = jnp.max(pv, axis=-1, keepdims=True)
        pv = jnp.where(pv == m, -jnp.inf, pv)
        outs.append(m)
    return jnp.concatenate(outs, axis=-1)


def _dense_topk_body(x_ref, wq_ref, bq_ref, caT_ref, cbT_ref,
                     fidx_ref, fw_ref):
    x = x_ref[...]
    q = jnp.dot(x, wq_ref[...], preferred_element_type=jnp.float32)
    q = q + bq_ref[...]
    qa = q[:, :_KEY_DIM]
    qb = q[:, _KEY_DIM:]
    sa = jnp.dot(qa, caT_ref[...], preferred_element_type=jnp.float32)
    sb = jnp.dot(qb, cbT_ref[...], preferred_element_type=jnp.float32)

    pa = _topk32_packed(sa)
    pb = _topk32_packed(sb)
    pai = lax.bitcast_convert_type(pa, jnp.int32)
    pbi = lax.bitcast_convert_type(pb, jnp.int32)
    ia = pai & 511
    ib = pbi & 511
    va = lax.bitcast_convert_type(pai & (~511), jnp.float32)
    vb = lax.bitcast_convert_type(pbi & (~511), jnp.float32)

    cv_parts, ci_parts = [], []
    for i, c in enumerate(_CNTS):
        cv_parts.append(va[:, i:i + 1] + vb[:, :c])
        ci_parts.append(ia[:, i:i + 1] * _SUB_KEYS + ib[:, :c])
    T = x.shape[0]
    pad = _CPAD - _NCAND
    cv_parts.append(jnp.full((T, pad), -3.0e38, jnp.float32))
    ci_parts.append(jnp.zeros((T, pad), jnp.int32))
    cv = jnp.concatenate(cv_parts, axis=-1)
    ci = jnp.concatenate(ci_parts, axis=-1)

    col = lax.broadcasted_iota(jnp.int32, cv.shape, 1)
    cvi = lax.bitcast_convert_type(cv, jnp.int32)
    cp = lax.bitcast_convert_type((cvi & (~127)) | col, jnp.float32)

    fvs, fis = [], []
    for _ in range(_TOP_K):
        m = jnp.max(cp, axis=-1, keepdims=True)
        eq = cp == m
        fis.append(jnp.sum(jnp.where(eq, ci, 0), axis=-1, keepdims=True))
        cp = jnp.where(eq, -jnp.inf, cp)
        fvs.append(m)
    fv_p = jnp.concatenate(fvs, axis=-1)
    fidx = jnp.concatenate(fis, axis=-1)
    fv = lax.bitcast_convert_type(
        lax.bitcast_convert_type(fv_p, jnp.int32) & (~127), jnp.float32)

    w = jnp.exp(fv - fv[:, :1])
    w = w / jnp.sum(w, axis=-1, keepdims=True)
    fidx_ref[...] = fidx
    fw_ref[...] = w


def _dense_topk(x, Wq, bq2, caT, cbT, tile):
    n = x.shape[0]
    grid = (n // tile,)
    return pl.pallas_call(
        _dense_topk_body,
        grid=grid,
        in_specs=[
            pl.BlockSpec((tile, _DIM), lambda i: (i, 0)),
            pl.BlockSpec((_DIM, 2 * _KEY_DIM), lambda i: (0, 0)),
            pl.BlockSpec((1, 2 * _KEY_DIM), lambda i: (0, 0)),
            pl.BlockSpec((_KEY_DIM, _SUB_KEYS), lambda i: (0, 0)),
            pl.BlockSpec((_KEY_DIM, _SUB_KEYS), lambda i: (0, 0)),
        ],
        out_specs=[
            pl.BlockSpec((tile, _TOP_K), lambda i: (i, 0)),
            pl.BlockSpec((tile, _TOP_K), lambda i: (i, 0)),
        ],
        out_shape=[
            jax.ShapeDtypeStruct((n, _TOP_K), jnp.int32),
            jax.ShapeDtypeStruct((n, _TOP_K), jnp.float32),
        ],
    )(x, Wq, bq2, caT, cbT)


def _finish_body(x_ref, g_ref, w_ref, wout_ref, bout_ref, lg_ref, lb_ref,
                 y_ref):
    w = w_ref[...]
    ws = w[:, 0:1] * g_ref[:, 0, :]
    for k in range(1, _TOP_K):
        ws = ws + w[:, k:k + 1] * g_ref[:, k, :]
    o = jnp.dot(ws, wout_ref[...], preferred_element_type=jnp.float32)
    h = x_ref[...] + o + bout_ref[...]
    mean = jnp.mean(h, axis=-1, keepdims=True)
    d = h - mean
    var = jnp.mean(d * d, axis=-1, keepdims=True)
    y_ref[...] = d * lax.rsqrt(var + 1e-5) * lg_ref[...] + lb_ref[...]


def _finish(x, gath, w, Wout, bout2, lg2, lb2, tile):
    n = x.shape[0]
    grid = (n // tile,)
    return pl.pallas_call(
        _finish_body,
        grid=grid,
        in_specs=[
            pl.BlockSpec((tile, _DIM), lambda i: (i, 0)),
            pl.BlockSpec((tile, _TOP_K, _VALUE_DIM), lambda i: (i, 0, 0)),
            pl.BlockSpec((tile, _TOP_K), lambda i: (i, 0)),
            pl.BlockSpec((_VALUE_DIM, _DIM), lambda i: (0, 0)),
            pl.BlockSpec((1, _DIM), lambda i: (0, 0)),
            pl.BlockSpec((1, _DIM), lambda i: (0, 0)),
            pl.BlockSpec((1, _DIM), lambda i: (0, 0)),
        ],
        out_specs=pl.BlockSpec((tile, _DIM), lambda i: (i, 0)),
        out_shape=jax.ShapeDtypeStruct((n, _DIM), jnp.float32),
    )(x, gath, w, Wout, bout2, lg2, lb2)


def kernel(tokens, tables, Wq, bq, codebook_a, codebook_b, values,
           Wout, bout, ln_g, ln_b):
    with jax.enable_x64(False):
        return _kernel_impl(tokens, tables, Wq, bq, codebook_a, codebook_b,
                            values, Wout, bout, ln_g, ln_b)


def _kernel_impl(tokens, tables, Wq, bq, codebook_a, codebook_b, values,
                 Wout, bout, ln_g, ln_b):
    Bs, Ts = tokens.shape
    n = Bs * Ts
    tok32 = tokens.astype(jnp.int32)
    tables_flat = tables.reshape(_NUM_TABLES * _BUCKETS, _EMBED_DIM)

    idx8 = _hash_indices(tok32)
    idx_embed = idx8.reshape(_NUM_TABLES, n).T.reshape(-1)

    emb = _sc_gather(tables_flat, idx_embed)
    x = emb.reshape(n, _NUM_TABLES * _EMBED_DIM)

    fidx, fw = _dense_topk(
        x, Wq.astype(jnp.float32), bq.reshape(1, -1),
        codebook_a.T, codebook_b.T, tile=128)

    gath = _sc_gather(values, fidx.reshape(-1))
    gath = gath.reshape(n, _TOP_K, _VALUE_DIM)

    y = _finish(x, gath, fw, Wout, bout.reshape(1, -1),
                ln_g.reshape(1, -1), ln_b.reshape(1, -1), tile=128)
    return y.reshape(Bs, Ts, _DIM)


# speedup vs baseline: 3.1114x; 1.0961x over previous
import functools

import jax
import jax.numpy as jnp
from jax import lax
from jax.experimental import pallas as pl
from jax.experimental.pallas import tpu as pltpu
from jax.experimental.pallas import tpu_sc as plsc

_HASH_PRIMES = [2654435761, 2246822519, 3266489917, 2028178513,
                1220703125, 1610612741, 805306457, 402653189]
_P16 = [p & 0xFFFF for p in _HASH_PRIMES]
_WINDOWS = (1, 2, 4, 8, 16, 32, 64, 128)

_DIM = 1024
_NUM_TABLES = 8
_BUCKETS = 65536
_EMBED_DIM = 128
_SUB_KEYS = 512
_TOP_K = 32
_KEY_DIM = 256
_VALUE_DIM = 256

_CNTS = [32 // (i + 1) for i in range(32)]
_NCAND = sum(_CNTS)
_CPAD = 128


def _hash_body(tok_ref, out_ref):
    t = tok_ref[...]
    z = jnp.zeros((t.shape[0], 1), jnp.int32)
    s = t
    acc = jnp.zeros_like(t)
    ti = 0
    for o in range(1, _WINDOWS[-1] + 1):
        s = jnp.concatenate([z, s[:, :-1]], axis=1)
        acc = acc ^ (s * _P16[(o - 1) % 8])
        if o == _WINDOWS[ti]:
            out_ref[ti] = (acc & 0xFFFF) + ti * _BUCKETS
            ti += 1


def _hash_indices(tok32):
    return pl.pallas_call(
        _hash_body,
        out_shape=jax.ShapeDtypeStruct((_NUM_TABLES,) + tok32.shape, jnp.int32),
    )(tok32)


def _sc_gather(table, idx_flat):
    V, D = table.shape
    NI = idx_flat.shape[0]
    NC, NW, CH = 2, 32, 128
    b_per_w = NI // NW
    n_chunks = b_per_w // CH
    mesh = plsc.VectorSubcoreMesh(core_axis_name="c", subcore_axis_name="s")

    @functools.partial(
        pl.kernel, mesh=mesh,
        out_type=jax.ShapeDtypeStruct((NI, D), jnp.float32),
        scratch_types=[
            pltpu.VMEM((CH,), jnp.int32),
            pltpu.VMEM((CH, D), jnp.float32),
            pltpu.SemaphoreType.DMA,
        ],
    )
    def k(table_hbm, idx_hbm, out_hbm, idx_v, rows_v, sem):
        wid = lax.axis_index("s") * jnp.int32(NC) + lax.axis_index("c")
        base = wid * jnp.int32(b_per_w)

        @pl.loop(0, n_chunks)
        def _(ci):
            off = base + ci * jnp.int32(CH)
            pltpu.sync_copy(idx_hbm.at[pl.ds(off, CH)], idx_v)
            pltpu.async_copy(table_hbm.at[idx_v], rows_v, sem).wait()
            pltpu.sync_copy(rows_v, out_hbm.at[pl.ds(off, CH)])

    return k(table, idx_flat)


def _topk32_packed(s):
    n_idx_bits_mask = 511
    lane = lax.broadcasted_iota(jnp.int32, s.shape, 1)
    si = lax.bitcast_convert_type(s, jnp.int32)
    pv = lax.bitcast_convert_type((si & (~n_idx_bits_mask)) | lane, jnp.float32)
    outs = []
    for _ in range(_TOP_K):
        m = jnp.max(pv, axis=-1, keepdims=True)
        pv = jnp.where(pv == m, -jnp.inf, pv)
        outs.append(m)
    return jnp.concatenate(outs, axis=-1)


def _dense_topk_body(x_ref, wq_ref, bq_ref, caT_ref, cbT_ref,
                     fidx_ref, fw_ref):
    x = x_ref[...]
    q = jnp.dot(x, wq_ref[...], preferred_element_type=jnp.float32)
    q = q + bq_ref[...]
    qa = q[:, :_KEY_DIM]
    qb = q[:, _KEY_DIM:]
    sa = jnp.dot(qa, caT_ref[...], preferred_element_type=jnp.float32)
    sb = jnp.dot(qb, cbT_ref[...], preferred_element_type=jnp.float32)

    pa = _topk32_packed(sa)
    pb = _topk32_packed(sb)
    pai = lax.bitcast_convert_type(pa, jnp.int32)
    pbi = lax.bitcast_convert_type(pb, jnp.int32)
    ia = pai & 511
    ib = pbi & 511
    va = lax.bitcast_convert_type(pai & (~511), jnp.float32)
    vb = lax.bitcast_convert_type(pbi & (~511), jnp.float32)

    cv_parts, ci_parts = [], []
    for i, c in enumerate(_CNTS):
        cv_parts.append(va[:, i:i + 1] + vb[:, :c])
        ci_parts.append(ia[:, i:i + 1] * _SUB_KEYS + ib[:, :c])
    T = x.shape[0]
    pad = _CPAD - _NCAND
    cv_parts.append(jnp.full((T, pad), -3.0e38, jnp.float32))
    ci_parts.append(jnp.zeros((T, pad), jnp.int32))
    cv = jnp.concatenate(cv_parts, axis=-1)
    ci = jnp.concatenate(ci_parts, axis=-1)

    col = lax.broadcasted_iota(jnp.int32, cv.shape, 1)
    cvi = lax.bitcast_convert_type(cv, jnp.int32)
    cp = lax.bitcast_convert_type((cvi & (~127)) | col, jnp.float32)

    fvs, fis = [], []
    for _ in range(_TOP_K):
        m = jnp.max(cp, axis=-1, keepdims=True)
        eq = cp == m
        fis.append(jnp.sum(jnp.where(eq, ci, 0), axis=-1, keepdims=True))
        cp = jnp.where(eq, -jnp.inf, cp)
        fvs.append(m)
    fv_p = jnp.concatenate(fvs, axis=-1)
    fidx = jnp.concatenate(fis, axis=-1)
    fv = lax.bitcast_convert_type(
        lax.bitcast_convert_type(fv_p, jnp.int32) & (~127), jnp.float32)

    w = jnp.exp(fv - fv[:, :1])
    w = w / jnp.sum(w, axis=-1, keepdims=True)
    fidx_ref[...] = fidx
    fw_ref[...] = w


def _dense_topk(x, Wq, bq2, caT, cbT, tile):
    n = x.shape[0]
    grid = (n // tile,)
    return pl.pallas_call(
        _dense_topk_body,
        grid=grid,
        in_specs=[
            pl.BlockSpec((tile, _DIM), lambda i: (i, 0)),
            pl.BlockSpec((_DIM, 2 * _KEY_DIM), lambda i: (0, 0)),
            pl.BlockSpec((1, 2 * _KEY_DIM), lambda i: (0, 0)),
            pl.BlockSpec((_KEY_DIM, _SUB_KEYS), lambda i: (0, 0)),
            pl.BlockSpec((_KEY_DIM, _SUB_KEYS), lambda i: (0, 0)),
        ],
        out_specs=[
            pl.BlockSpec((tile, _TOP_K), lambda i: (i, 0)),
            pl.BlockSpec((tile, _TOP_K), lambda i: (i, 0)),
        ],
        out_shape=[
            jax.ShapeDtypeStruct((n, _TOP_K), jnp.int32),
            jax.ShapeDtypeStruct((n, _TOP_K), jnp.float32),
        ],
    )(x, Wq, bq2, caT, cbT)


def _finish_body(x_ref, g_ref, w_ref, wout_ref, bout_ref, lg_ref, lb_ref,
                 y_ref):
    w = w_ref[...]
    ws = w[:, 0:1] * g_ref[:, 0, :]
    for k in range(1, _TOP_K):
        ws = ws + w[:, k:k + 1] * g_ref[:, k, :]
    o = jnp.dot(ws, wout_ref[...], preferred_element_type=jnp.float32)
    h = x_ref[...] + o + bout_ref[...]
    mean = jnp.mean(h, axis=-1, keepdims=True)
    d = h - mean
    var = jnp.mean(d * d, axis=-1, keepdims=True)
    y_ref[...] = d * lax.rsqrt(var + 1e-5) * lg_ref[...] + lb_ref[...]


def _finish(x, gath, w, Wout, bout2, lg2, lb2, tile):
    n = x.shape[0]
    grid = (n // tile,)
    return pl.pallas_call(
        _finish_body,
        grid=grid,
        in_specs=[
            pl.BlockSpec((tile, _DIM), lambda i: (i, 0)),
            pl.BlockSpec((tile, _TOP_K, _VALUE_DIM), lambda i: (i, 0, 0)),
            pl.BlockSpec((tile, _TOP_K), lambda i: (i, 0)),
            pl.BlockSpec((_VALUE_DIM, _DIM), lambda i: (0, 0)),
            pl.BlockSpec((1, _DIM), lambda i: (0, 0)),
            pl.BlockSpec((1, _DIM), lambda i: (0, 0)),
            pl.BlockSpec((1, _DIM), lambda i: (0, 0)),
        ],
        out_specs=pl.BlockSpec((tile, _DIM), lambda i: (i, 0)),
        out_shape=jax.ShapeDtypeStruct((n, _DIM), jnp.float32),
    )(x, gath, w, Wout, bout2, lg2, lb2)


def kernel(tokens, tables, Wq, bq, codebook_a, codebook_b, values,
           Wout, bout, ln_g, ln_b):
    with jax.enable_x64(False):
        return _kernel_impl(tokens, tables, Wq, bq, codebook_a, codebook_b,
                            values, Wout, bout, ln_g, ln_b)


def _kernel_impl(tokens, tables, Wq, bq, codebook_a, codebook_b, values,
                 Wout, bout, ln_g, ln_b):
    Bs, Ts = tokens.shape
    n = Bs * Ts
    tok32 = tokens.astype(jnp.int32)
    tables_flat = tables.reshape(_NUM_TABLES * _BUCKETS, _EMBED_DIM)

    idx8 = _hash_indices(tok32)
    idx_embed = idx8.reshape(_NUM_TABLES, n).T.reshape(-1)

    emb = _sc_gather(tables_flat, idx_embed)
    x = emb.reshape(n, _NUM_TABLES * _EMBED_DIM)

    Wq32 = Wq.astype(jnp.float32)
    bq2 = bq.reshape(1, -1)
    caT = codebook_a.T
    cbT = codebook_b.T
    bout2 = bout.reshape(1, -1)
    lg2 = ln_g.reshape(1, -1)
    lb2 = ln_b.reshape(1, -1)

    n_chunks = 4
    cs = n // n_chunks
    ys = []
    for c in range(n_chunks):
        xc = lax.slice_in_dim(x, c * cs, (c + 1) * cs, axis=0)
        fidx, fw = _dense_topk(xc, Wq32, bq2, caT, cbT, tile=128)
        gath = _sc_gather(values, fidx.reshape(-1))
        gath = gath.reshape(cs, _TOP_K, _VALUE_DIM)
        ys.append(_finish(xc, gath, fw, Wout, bout2, lg2, lb2, tile=128))
    y = jnp.concatenate(ys, axis=0)
    return y.reshape(Bs, Ts, _DIM)


# speedup vs baseline: 3.7440x; 1.2033x over previous
import functools

import jax
import jax.numpy as jnp
from jax import lax
from jax.experimental import pallas as pl
from jax.experimental.pallas import tpu as pltpu
from jax.experimental.pallas import tpu_sc as plsc

_HASH_PRIMES = [2654435761, 2246822519, 3266489917, 2028178513,
                1220703125, 1610612741, 805306457, 402653189]
_P16 = [p & 0xFFFF for p in _HASH_PRIMES]
_WINDOWS = (1, 2, 4, 8, 16, 32, 64, 128)

_DIM = 1024
_NUM_TABLES = 8
_BUCKETS = 65536
_EMBED_DIM = 128
_SUB_KEYS = 512
_TOP_K = 32
_KEY_DIM = 256
_VALUE_DIM = 256

_CNTS = [32 // (i + 1) for i in range(32)]
_NCAND = sum(_CNTS)
_CPAD = 128


def _hash_body(tok_ref, out_ref):
    t = tok_ref[...]

    def shift(a, k):
        z = jnp.zeros((a.shape[0], k), jnp.int32)
        return jnp.concatenate([z, a[:, :-k]], axis=1)

    cur = [shift(t, c + 1) * _P16[c] for c in range(8)]
    partial = [None] * 8
    done = [0] * 8
    for ti, w in enumerate(_WINDOWS):
        for c in range(8):
            while done[c] * 8 + c + 1 <= w:
                partial[c] = cur[c] if partial[c] is None else partial[c] ^ cur[c]
                done[c] += 1
                if done[c] * 8 + c + 1 <= _WINDOWS[-1]:
                    cur[c] = shift(cur[c], 8)
        h = partial[0]
        for p in partial[1:]:
            if p is not None:
                h = h ^ p
        out_ref[ti] = (h & 0xFFFF) + ti * _BUCKETS


def _hash_indices(tok32):
    return pl.pallas_call(
        _hash_body,
        out_shape=jax.ShapeDtypeStruct((_NUM_TABLES,) + tok32.shape, jnp.int32),
    )(tok32)


def _sc_gather(table, idx_flat):
    V, D = table.shape
    NI = idx_flat.shape[0]
    NC, NW, CH = 2, 32, 128
    b_per_w = NI // NW
    n_chunks = b_per_w // CH
    mesh = plsc.VectorSubcoreMesh(core_axis_name="c", subcore_axis_name="s")

    @functools.partial(
        pl.kernel, mesh=mesh,
        out_type=jax.ShapeDtypeStruct((NI, D), jnp.float32),
        scratch_types=[
            pltpu.VMEM((CH,), jnp.int32),
            pltpu.VMEM((CH, D), jnp.float32),
            pltpu.SemaphoreType.DMA,
        ],
    )
    def k(table_hbm, idx_hbm, out_hbm, idx_v, rows_v, sem):
        wid = lax.axis_index("s") * jnp.int32(NC) + lax.axis_index("c")
        base = wid * jnp.int32(b_per_w)

        @pl.loop(0, n_chunks)
        def _(ci):
            off = base + ci * jnp.int32(CH)
            pltpu.sync_copy(idx_hbm.at[pl.ds(off, CH)], idx_v)
            pltpu.async_copy(table_hbm.at[idx_v], rows_v, sem).wait()
            pltpu.sync_copy(rows_v, out_hbm.at[pl.ds(off, CH)])

    return k(table, idx_flat)


def _topk32_packed(s):
    lane = lax.broadcasted_iota(jnp.int32, s.shape, 1)
    si = lax.bitcast_convert_type(s, jnp.int32)
    pv = lax.bitcast_convert_type((si & (~511)) | lane, jnp.float32)
    outs = []
    for _ in range(_TOP_K):
        f = jnp.maximum(jnp.maximum(pv[:, 0:128], pv[:, 128:256]),
                        jnp.maximum(pv[:, 256:384], pv[:, 384:512]))
        m = jnp.max(f, axis=-1, keepdims=True)
        pv = jnp.where(pv == m, -jnp.inf, pv)
        outs.append(m)
    return jnp.concatenate(outs, axis=-1)


def _dense_topk_body(x_ref, wq_ref, bq_ref, caT_ref, cbT_ref, ij_ref,
                     fidx_ref, fw_ref):
    x = x_ref[...]
    q = jnp.dot(x, wq_ref[...], preferred_element_type=jnp.float32)
    q = q + bq_ref[...]
    qa = q[:, :_KEY_DIM]
    qb = q[:, _KEY_DIM:]
    sa = jnp.dot(qa, caT_ref[...], preferred_element_type=jnp.float32)
    sb = jnp.dot(qb, cbT_ref[...], preferred_element_type=jnp.float32)

    pa = _topk32_packed(sa)
    pb = _topk32_packed(sb)
    pai = lax.bitcast_convert_type(pa, jnp.int32)
    pbi = lax.bitcast_convert_type(pb, jnp.int32)
    ia = pai & 511
    ib = pbi & 511
    va = lax.bitcast_convert_type(pai & (~511), jnp.float32)
    vb = lax.bitcast_convert_type(pbi & (~511), jnp.float32)

    T = x.shape[0]
    I = jnp.broadcast_to(ij_ref[0:1, :], (T, _CPAD))
    J = jnp.broadcast_to(ij_ref[1:2, :], (T, _CPAD))
    va_g = jnp.take_along_axis(va, I, axis=-1)
    vb_g = jnp.take_along_axis(vb, J, axis=-1)
    ia_g = jnp.take_along_axis(ia, I, axis=-1)
    ib_g = jnp.take_along_axis(ib, J, axis=-1)
    cv = va_g + vb_g
    ci = ia_g * _SUB_KEYS + ib_g

    col = lax.broadcasted_iota(jnp.int32, cv.shape, 1)
    valid = col < _NCAND
    cv = jnp.where(valid, cv, -3.0e38)
    cvi = lax.bitcast_convert_type(cv, jnp.int32)
    cp = lax.bitcast_convert_type((cvi & (~127)) | col, jnp.float32)

    fvs = []
    for _ in range(_TOP_K):
        m = jnp.max(cp, axis=-1, keepdims=True)
        cp = jnp.where(cp == m, -jnp.inf, cp)
        fvs.append(m)
    fv_p = lax.bitcast_convert_type(
        jnp.concatenate(fvs, axis=-1), jnp.int32)
    cols = fv_p & 127
    fidx = jnp.take_along_axis(ci, cols, axis=-1)
    fv = lax.bitcast_convert_type(fv_p & (~127), jnp.float32)

    w = jnp.exp(fv - fv[:, :1])
    w = w / jnp.sum(w, axis=-1, keepdims=True)
    fidx_ref[...] = fidx
    fw_ref[...] = w


def _staircase_ij():
    i_pat, j_pat = [], []
    for i, c in enumerate(_CNTS):
        i_pat.extend([i] * c)
        j_pat.extend(range(c))
    i_pat.extend([0] * (_CPAD - _NCAND))
    j_pat.extend([0] * (_CPAD - _NCAND))
    import numpy as _np
    return jnp.asarray(_np.array([i_pat, j_pat], _np.int32))


def _dense_topk(x, Wq, bq2, caT, cbT, ij, tile):
    n = x.shape[0]
    grid = (n // tile,)
    return pl.pallas_call(
        _dense_topk_body,
        grid=grid,
        in_specs=[
            pl.BlockSpec((tile, _DIM), lambda i: (i, 0)),
            pl.BlockSpec((_DIM, 2 * _KEY_DIM), lambda i: (0, 0)),
            pl.BlockSpec((1, 2 * _KEY_DIM), lambda i: (0, 0)),
            pl.BlockSpec((_KEY_DIM, _SUB_KEYS), lambda i: (0, 0)),
            pl.BlockSpec((_KEY_DIM, _SUB_KEYS), lambda i: (0, 0)),
            pl.BlockSpec((2, _CPAD), lambda i: (0, 0)),
        ],
        out_specs=[
            pl.BlockSpec((tile, _TOP_K), lambda i: (i, 0)),
            pl.BlockSpec((tile, _TOP_K), lambda i: (i, 0)),
        ],
        out_shape=[
            jax.ShapeDtypeStruct((n, _TOP_K), jnp.int32),
            jax.ShapeDtypeStruct((n, _TOP_K), jnp.float32),
        ],
    )(x, Wq, bq2, caT, cbT, ij)


def _finish_body(x_ref, g_ref, w_ref, wout_ref, bout_ref, lg_ref, lb_ref,
                 y_ref):
    w = w_ref[...]
    ws = w[:, 0:1] * g_ref[:, 0, :]
    for k in range(1, _TOP_K):
        ws = ws + w[:, k:k + 1] * g_ref[:, k, :]
    o = jnp.dot(ws, wout_ref[...], preferred_element_type=jnp.float32)
    h = x_ref[...] + o + bout_ref[...]
    mean = jnp.mean(h, axis=-1, keepdims=True)
    d = h - mean
    var = jnp.mean(d * d, axis=-1, keepdims=True)
    y_ref[...] = d * lax.rsqrt(var + 1e-5) * lg_ref[...] + lb_ref[...]


def _finish(x, gath, w, Wout, bout2, lg2, lb2, tile):
    n = x.shape[0]
    grid = (n // tile,)
    return pl.pallas_call(
        _finish_body,
        grid=grid,
        in_specs=[
            pl.BlockSpec((tile, _DIM), lambda i: (i, 0)),
            pl.BlockSpec((tile, _TOP_K, _VALUE_DIM), lambda i: (i, 0, 0)),
            pl.BlockSpec((tile, _TOP_K), lambda i: (i, 0)),
            pl.BlockSpec((_VALUE_DIM, _DIM), lambda i: (0, 0)),
            pl.BlockSpec((1, _DIM), lambda i: (0, 0)),
            pl.BlockSpec((1, _DIM), lambda i: (0, 0)),
            pl.BlockSpec((1, _DIM), lambda i: (0, 0)),
        ],
        out_specs=pl.BlockSpec((tile, _DIM), lambda i: (i, 0)),
        out_shape=jax.ShapeDtypeStruct((n, _DIM), jnp.float32),
    )(x, gath, w, Wout, bout2, lg2, lb2)


def kernel(tokens, tables, Wq, bq, codebook_a, codebook_b, values,
           Wout, bout, ln_g, ln_b):
    with jax.enable_x64(False):
        return _kernel_impl(tokens, tables, Wq, bq, codebook_a, codebook_b,
                            values, Wout, bout, ln_g, ln_b)


def _kernel_impl(tokens, tables, Wq, bq, codebook_a, codebook_b, values,
                 Wout, bout, ln_g, ln_b):
    Bs, Ts = tokens.shape
    n = Bs * Ts
    tok32 = tokens.astype(jnp.int32)
    tables_flat = tables.reshape(_NUM_TABLES * _BUCKETS, _EMBED_DIM)

    idx8 = _hash_indices(tok32)
    idx_embed = idx8.reshape(_NUM_TABLES, n).T.reshape(-1)

    emb = _sc_gather(tables_flat, idx_embed)
    x = emb.reshape(n, _NUM_TABLES * _EMBED_DIM)

    Wq32 = Wq.astype(jnp.float32)
    bq2 = bq.reshape(1, -1)
    caT = codebook_a.T
    cbT = codebook_b.T
    bout2 = bout.reshape(1, -1)
    lg2 = ln_g.reshape(1, -1)
    lb2 = ln_b.reshape(1, -1)
    ij = _staircase_ij()

    n_chunks = 4
    cs = n // n_chunks
    ys = []
    for c in range(n_chunks):
        xc = lax.slice_in_dim(x, c * cs, (c + 1) * cs, axis=0)
        fidx, fw = _dense_topk(xc, Wq32, bq2, caT, cbT, ij, tile=128)
        gath = _sc_gather(values, fidx.reshape(-1))
        gath = gath.reshape(cs, _TOP_K, _VALUE_DIM)
        ys.append(_finish(xc, gath, fw, Wout, bout2, lg2, lb2, tile=128))
    y = jnp.concatenate(ys, axis=0)
    return y.reshape(Bs, Ts, _DIM)


# speedup vs baseline: 4.6645x; 1.2459x over previous
import functools

import jax
import jax.numpy as jnp
from jax import lax
from jax.experimental import pallas as pl
from jax.experimental.pallas import tpu as pltpu
from jax.experimental.pallas import tpu_sc as plsc

_HASH_PRIMES = [2654435761, 2246822519, 3266489917, 2028178513,
                1220703125, 1610612741, 805306457, 402653189]
_P16 = [p & 0xFFFF for p in _HASH_PRIMES]
_WINDOWS = (1, 2, 4, 8, 16, 32, 64, 128)

_DIM = 1024
_NUM_TABLES = 8
_BUCKETS = 65536
_EMBED_DIM = 128
_SUB_KEYS = 512
_TOP_K = 32
_KEY_DIM = 256
_VALUE_DIM = 256

_CNTS = [32 // (i + 1) for i in range(32)]
_NCAND = sum(_CNTS)
_CPAD = 128


def _hash_body(tok_ref, out_ref):
    t = tok_ref[...]

    def shift(a, k):
        z = jnp.zeros((a.shape[0], k), jnp.int32)
        return jnp.concatenate([z, a[:, :-k]], axis=1)

    cur = [shift(t, c + 1) * _P16[c] for c in range(8)]
    partial = [None] * 8
    done = [0] * 8
    for ti, w in enumerate(_WINDOWS):
        for c in range(8):
            while done[c] * 8 + c + 1 <= w:
                partial[c] = cur[c] if partial[c] is None else partial[c] ^ cur[c]
                done[c] += 1
                if done[c] * 8 + c + 1 <= _WINDOWS[-1]:
                    cur[c] = shift(cur[c], 8)
        h = partial[0]
        for p in partial[1:]:
            if p is not None:
                h = h ^ p
        out_ref[ti] = (h & 0xFFFF) + ti * _BUCKETS


def _hash_indices(tok32):
    return pl.pallas_call(
        _hash_body,
        out_shape=jax.ShapeDtypeStruct((_NUM_TABLES,) + tok32.shape, jnp.int32),
    )(tok32)


def _sc_gather(table, idx_flat):
    V, D = table.shape
    NI = idx_flat.shape[0]
    NC, NW, CH = 2, 32, 128
    b_per_w = NI // NW
    n_chunks = b_per_w // CH
    mesh = plsc.VectorSubcoreMesh(core_axis_name="c", subcore_axis_name="s")

    @functools.partial(
        pl.kernel, mesh=mesh,
        out_type=jax.ShapeDtypeStruct((NI, D), jnp.float32),
        scratch_types=[
            pltpu.VMEM((CH,), jnp.int32),
            pltpu.VMEM((CH, D), jnp.float32),
            pltpu.SemaphoreType.DMA,
        ],
    )
    def k(table_hbm, idx_hbm, out_hbm, idx_v, rows_v, sem):
        wid = lax.axis_index("s") * jnp.int32(NC) + lax.axis_index("c")
        base = wid * jnp.int32(b_per_w)

        @pl.loop(0, n_chunks)
        def _(ci):
            off = base + ci * jnp.int32(CH)
            pltpu.sync_copy(idx_hbm.at[pl.ds(off, CH)], idx_v)
            pltpu.async_copy(table_hbm.at[idx_v], rows_v, sem).wait()
            pltpu.sync_copy(rows_v, out_hbm.at[pl.ds(off, CH)])

    return k(table, idx_flat)


def _topk32_packed_multi(ss):
    pvs = []
    for s in ss:
        lane = lax.broadcasted_iota(jnp.int32, s.shape, 1)
        si = lax.bitcast_convert_type(s, jnp.int32)
        pvs.append(lax.bitcast_convert_type((si & (~511)) | lane, jnp.float32))
    outs = [[] for _ in ss]
    for _ in range(_TOP_K):
        for a, pv in enumerate(pvs):
            f = jnp.maximum(jnp.maximum(pv[:, 0:128], pv[:, 128:256]),
                            jnp.maximum(pv[:, 256:384], pv[:, 384:512]))
            m = jnp.max(f, axis=-1, keepdims=True)
            pvs[a] = jnp.where(pv == m, -jnp.inf, pv)
            outs[a].append(m)
    return [jnp.concatenate(o, axis=-1) for o in outs]


def _dense_topk_body(x_ref, wq_ref, bq_ref, caT_ref, cbT_ref, ij_ref,
                     fidx_ref, fw_ref):
    x = x_ref[...]
    q = jnp.dot(x, wq_ref[...], preferred_element_type=jnp.float32)
    q = q + bq_ref[...]
    qa = q[:, :_KEY_DIM]
    qb = q[:, _KEY_DIM:]
    sa = jnp.dot(qa, caT_ref[...], preferred_element_type=jnp.float32)
    sb = jnp.dot(qb, cbT_ref[...], preferred_element_type=jnp.float32)

    pa, pb = _topk32_packed_multi([sa, sb])
    pai = lax.bitcast_convert_type(pa, jnp.int32)
    pbi = lax.bitcast_convert_type(pb, jnp.int32)
    ia = pai & 511
    ib = pbi & 511
    va = lax.bitcast_convert_type(pai & (~511), jnp.float32)
    vb = lax.bitcast_convert_type(pbi & (~511), jnp.float32)

    T = x.shape[0]
    I = jnp.broadcast_to(ij_ref[0:1, :], (T, _CPAD))
    J = jnp.broadcast_to(ij_ref[1:2, :], (T, _CPAD))
    va_g = jnp.take_along_axis(va, I, axis=-1)
    vb_g = jnp.take_along_axis(vb, J, axis=-1)
    ia_g = jnp.take_along_axis(ia, I, axis=-1)
    ib_g = jnp.take_along_axis(ib, J, axis=-1)
    cv = va_g + vb_g
    ci = ia_g * _SUB_KEYS + ib_g

    col = lax.broadcasted_iota(jnp.int32, cv.shape, 1)
    valid = col < _NCAND
    cv = jnp.where(valid, cv, -3.0e38)
    cvi = lax.bitcast_convert_type(cv, jnp.int32)
    cp = lax.bitcast_convert_type((cvi & (~127)) | col, jnp.float32)

    fvs = []
    for _ in range(_TOP_K):
        m = jnp.max(cp, axis=-1, keepdims=True)
        cp = jnp.where(cp == m, -jnp.inf, cp)
        fvs.append(m)
    fv_p = lax.bitcast_convert_type(
        jnp.concatenate(fvs, axis=-1), jnp.int32)
    cols = fv_p & 127
    fidx = jnp.take_along_axis(ci, cols, axis=-1)
    fv = lax.bitcast_convert_type(fv_p & (~127), jnp.float32)

    w = jnp.exp(fv - fv[:, :1])
    w = w / jnp.sum(w, axis=-1, keepdims=True)
    fidx_ref[...] = fidx
    fw_ref[...] = w


def _staircase_ij():
    i_pat, j_pat = [], []
    for i, c in enumerate(_CNTS):
        i_pat.extend([i] * c)
        j_pat.extend(range(c))
    i_pat.extend([0] * (_CPAD - _NCAND))
    j_pat.extend([0] * (_CPAD - _NCAND))
    import numpy as _np
    return jnp.asarray(_np.array([i_pat, j_pat], _np.int32))


def _dense_topk(x, Wq, bq2, caT, cbT, ij, tile):
    n = x.shape[0]
    grid = (n // tile,)
    return pl.pallas_call(
        _dense_topk_body,
        grid=grid,
        in_specs=[
            pl.BlockSpec((tile, _DIM), lambda i: (i, 0)),
            pl.BlockSpec((_DIM, 2 * _KEY_DIM), lambda i: (0, 0)),
            pl.BlockSpec((1, 2 * _KEY_DIM), lambda i: (0, 0)),
            pl.BlockSpec((_KEY_DIM, _SUB_KEYS), lambda i: (0, 0)),
            pl.BlockSpec((_KEY_DIM, _SUB_KEYS), lambda i: (0, 0)),
            pl.BlockSpec((2, _CPAD), lambda i: (0, 0)),
        ],
        out_specs=[
            pl.BlockSpec((tile, _TOP_K), lambda i: (i, 0)),
            pl.BlockSpec((tile, _TOP_K), lambda i: (i, 0)),
        ],
        out_shape=[
            jax.ShapeDtypeStruct((n, _TOP_K), jnp.int32),
            jax.ShapeDtypeStruct((n, _TOP_K), jnp.float32),
        ],
    )(x, Wq, bq2, caT, cbT, ij)


def _finish_body(x_ref, g_ref, w_ref, wout_ref, bout_ref, lg_ref, lb_ref,
                 y_ref):
    w = w_ref[...]
    ws = w[:, 0:1] * g_ref[:, 0, :]
    for k in range(1, _TOP_K):
        ws = ws + w[:, k:k + 1] * g_ref[:, k, :]
    o = jnp.dot(ws, wout_ref[...], preferred_element_type=jnp.float32)
    h = x_ref[...] + o + bout_ref[...]
    mean = jnp.mean(h, axis=-1, keepdims=True)
    d = h - mean
    var = jnp.mean(d * d, axis=-1, keepdims=True)
    y_ref[...] = d * lax.rsqrt(var + 1e-5) * lg_ref[...] + lb_ref[...]


def _finish(x, gath, w, Wout, bout2, lg2, lb2, tile):
    n = x.shape[0]
    grid = (n // tile,)
    return pl.pallas_call(
        _finish_body,
        grid=grid,
        in_specs=[
            pl.BlockSpec((tile, _DIM), lambda i: (i, 0)),
            pl.BlockSpec((tile, _TOP_K, _VALUE_DIM), lambda i: (i, 0, 0)),
            pl.BlockSpec((tile, _TOP_K), lambda i: (i, 0)),
            pl.BlockSpec((_VALUE_DIM, _DIM), lambda i: (0, 0)),
            pl.BlockSpec((1, _DIM), lambda i: (0, 0)),
            pl.BlockSpec((1, _DIM), lambda i: (0, 0)),
            pl.BlockSpec((1, _DIM), lambda i: (0, 0)),
        ],
        out_specs=pl.BlockSpec((tile, _DIM), lambda i: (i, 0)),
        out_shape=jax.ShapeDtypeStruct((n, _DIM), jnp.float32),
    )(x, gath, w, Wout, bout2, lg2, lb2)


def kernel(tokens, tables, Wq, bq, codebook_a, codebook_b, values,
           Wout, bout, ln_g, ln_b):
    with jax.enable_x64(False):
        return _kernel_impl(tokens, tables, Wq, bq, codebook_a, codebook_b,
                            values, Wout, bout, ln_g, ln_b)


def _kernel_impl(tokens, tables, Wq, bq, codebook_a, codebook_b, values,
                 Wout, bout, ln_g, ln_b):
    Bs, Ts = tokens.shape
    n = Bs * Ts
    tok32 = tokens.astype(jnp.int32)
    tables_flat = tables.reshape(_NUM_TABLES * _BUCKETS, _EMBED_DIM)

    idx8 = _hash_indices(tok32)
    idx_embed = idx8.reshape(_NUM_TABLES, n).T.reshape(-1)

    emb = _sc_gather(tables_flat, idx_embed)
    x = emb.reshape(n, _NUM_TABLES * _EMBED_DIM)

    Wq32 = Wq.astype(jnp.float32)
    bq2 = bq.reshape(1, -1)
    caT = codebook_a.T
    cbT = codebook_b.T
    bout2 = bout.reshape(1, -1)
    lg2 = ln_g.reshape(1, -1)
    lb2 = ln_b.reshape(1, -1)
    ij = _staircase_ij()

    n_chunks = 4
    cs = n // n_chunks
    ys = []
    for c in range(n_chunks):
        xc = lax.slice_in_dim(x, c * cs, (c + 1) * cs, axis=0)
        fidx, fw = _dense_topk(xc, Wq32, bq2, caT, cbT, ij, tile=512)
        gath = _sc_gather(values, fidx.reshape(-1))
        gath = gath.reshape(cs, _TOP_K, _VALUE_DIM)
        ys.append(_finish(xc, gath, fw, Wout, bout2, lg2, lb2, tile=128))
    y = jnp.concatenate(ys, axis=0)
    return y.reshape(Bs, Ts, _DIM)
